# Initial kernel scaffold; baseline (speedup 1.0000x reference)
import jax
import jax.numpy as jnp
from jax.experimental import pallas as pl


def kernel(x, edge_index, W_in, b_in, W_gat, att_src, att_dst, b_gat, bn_gamma, bn_beta, W1, b1, W2, b2):
    raise NotImplementedError("write your pallas kernel here")


# speedup vs baseline: 13.0218x; 13.0218x over previous
import jax
import jax.numpy as jnp
from jax import lax
from jax.experimental import pallas as pl
from jax.experimental.pallas import tpu as pltpu
from jax.experimental.pallas import tpu_sc as plsc

_N = 10000
_E = 160000
_ETOT = _E + _N
_HID = 128
_HEADS = 8
_HH = _HEADS * _HID
_L = 4
_ALPHA = 0.1

_NC = 2
_NS = 16
_CHUNK = 64
_CPT = 84
_EPT = _CHUNK * _CPT
_EPAD = _EPT * _NC * _NS
_NP = 10240
_RPT = _NP // _NS

_f32 = jnp.float32


def _elu(v):
    return jnp.where(v > 0.0, v, jnp.exp(jnp.minimum(v, 0.0)) - 1.0)


def _vgather16(v, idx):
    dn = lax.GatherDimensionNumbers(
        offset_dims=(), collapsed_slice_dims=(0,), start_index_map=(0,))
    return lax.gather(v, idx[:, None], dn, slice_sizes=(1,),
                      mode=lax.GatherScatterMode.PROMISE_IN_BOUNDS)


def _mesh():
    return plsc.VectorSubcoreMesh(core_axis_name="c", subcore_axis_name="s",
                                  num_cores=_NC, num_subcores=_NS)


def _tc_in_body(x_ref, w_ref, b_ref, o_ref):
    o_ref[...] = _elu(
        jnp.dot(x_ref[...], w_ref[...], preferred_element_type=_f32)
        + b_ref[...])


def _tc_in(x, W, b):
    return pl.pallas_call(
        _tc_in_body,
        out_shape=jax.ShapeDtypeStruct((_N, _HID), _f32),
    )(x, W, b.reshape(1, _HID))


def _tc_pre_body(h_ref, wlo_ref, whi_ref, wt_ref, lo_ref, hi_ref, t_ref):
    h = h_ref[...]
    lo_ref[...] = jnp.dot(h, wlo_ref[...], preferred_element_type=_f32)
    hi_ref[...] = jnp.dot(h, whi_ref[...], preferred_element_type=_f32)
    t_ref[...] = jnp.dot(h, wt_ref[...], preferred_element_type=_f32)


def _tc_pre(h, Wlo, Whi, Wt):
    br = 1000
    hh2 = _HH // 2
    return pl.pallas_call(
        _tc_pre_body,
        grid=(_N // br,),
        in_specs=[pl.BlockSpec((br, _HID), lambda i: (i, 0)),
                  pl.BlockSpec((_HID, hh2), lambda i: (0, 0)),
                  pl.BlockSpec((_HID, hh2), lambda i: (0, 0)),
                  pl.BlockSpec((_HID, 16), lambda i: (0, 0))],
        out_specs=[pl.BlockSpec((br, hh2), lambda i: (i, 0)),
                   pl.BlockSpec((br, hh2), lambda i: (i, 0)),
                   pl.BlockSpec((br, 16), lambda i: (i, 0))],
        out_shape=[jax.ShapeDtypeStruct((_N, hh2), _f32),
                   jax.ShapeDtypeStruct((_N, hh2), _f32),
                   jax.ShapeDtypeStruct((_N, 16), _f32)],
    )(h, Wlo, Whi, Wt)


def _tc_post_body(p0l_ref, p1l_ref, p0h_ref, p1h_ref, hprev_ref,
                  b_ref, ga_ref, be_ref, o_ref):
    hw = _HID // 2
    for j, (a_ref, b2_ref) in enumerate([(p0l_ref, p1l_ref),
                                         (p0h_ref, p1h_ref)]):
        sl = slice(j * hw, (j + 1) * hw)
        g = a_ref[0:_N, :] + b2_ref[0:_N, :] + b_ref[:, sl]
        mu = jnp.mean(g, axis=0, keepdims=True)
        xc = g - mu
        var = jnp.mean(xc * xc, axis=0, keepdims=True)
        gn = xc * lax.rsqrt(var + 1e-5) * ga_ref[:, sl] + be_ref[:, sl]
        o_ref[:, sl] = _elu((1.0 - _ALPHA) * gn
                            + _ALPHA * hprev_ref[:, sl])


def _tc_post(p0l, p1l, p0h, p1h, hprev, b, gamma, beta):
    return pl.pallas_call(
        _tc_post_body,
        out_shape=jax.ShapeDtypeStruct((_N, _HID), _f32),
    )(p0l, p1l, p0h, p1h, hprev, b.reshape(1, _HID),
      gamma.reshape(1, _HID), beta.reshape(1, _HID))


def _tc_head_body(h_ref, w1_ref, b1_ref, w2_ref, b2_ref, o_ref):
    z = _elu(jnp.dot(h_ref[...], w1_ref[...], preferred_element_type=_f32)
             + b1_ref[...])
    o_ref[...] = jnp.dot(z, w2_ref[...], preferred_element_type=_f32) \
        + b2_ref[...]


def _tc_head(h, W1, b1, W2p, b2p):
    return pl.pallas_call(
        _tc_head_body,
        out_shape=jax.ShapeDtypeStruct((_N, _HID), _f32),
    )(h, W1, b1.reshape(1, _HID // 2), W2p, b2p.reshape(1, _HID))


def _sc_a_body(t_hbm, src_hbm, dst_hbm, ex_hbm, den0_hbm, den1_hbm,
               sidx, didx, ts, td, exb, zbuf, den_sh, sem):
    c = lax.axis_index("c")
    s = lax.axis_index("s")
    wid = c * _NS + s

    def zrow(r, carry):
        zbuf[r, :] = jnp.zeros((16,), _f32)
        return carry
    lax.fori_loop(0, _RPT, zrow, 0)
    pltpu.sync_copy(zbuf, den_sh.at[pl.ds(s * _RPT, _RPT), :])
    plsc.subcore_barrier()

    def chunk_body(k, carry):
        base = wid * _EPT + k * _CHUNK
        pltpu.sync_copy(src_hbm.at[pl.ds(base, _CHUNK)], sidx)
        pltpu.sync_copy(dst_hbm.at[pl.ds(base, _CHUNK)], didx)
        pltpu.async_copy(t_hbm.at[sidx], ts, sem).wait()
        pltpu.async_copy(t_hbm.at[didx], td, sem).wait()

        def edge(e, c2):
            lane = lax.iota(jnp.int32, 16)
            shift8 = jnp.minimum(lane + 8, 15)
            srow = ts[e, :]
            drow = _vgather16(td[e, :], shift8)
            esum = srow + drow
            lr = jnp.where(esum >= 0.0, esum, 0.2 * esum)
            ex = jnp.exp(lr)
            bound = jnp.where(base + e < _ETOT, 8, 0)
            bv = jnp.full((16,), bound, jnp.int32)
            exb[e, :] = jnp.where(lane < bv, ex, 0.0)
            return c2
        lax.fori_loop(0, _CHUNK, edge, 0)
        pltpu.sync_copy(exb, ex_hbm.at[pl.ds(base, _CHUNK), :])
        pltpu.sync_copy(exb, den_sh.at[didx], add=True)
        return carry
    lax.fori_loop(0, _CPT, chunk_body, 0)
    plsc.subcore_barrier()

    @pl.when(c == 0)
    def _():
        pltpu.sync_copy(den_sh.at[pl.ds(s * _RPT, _RPT), :],
                        den0_hbm.at[pl.ds(s * _RPT, _RPT), :])

    @pl.when(c == 1)
    def _():
        pltpu.sync_copy(den_sh.at[pl.ds(s * _RPT, _RPT), :],
                        den1_hbm.at[pl.ds(s * _RPT, _RPT), :])


def _sc_pass_a(T, src, dst):
    return pl.kernel(
        _sc_a_body,
        out_type=[jax.ShapeDtypeStruct((_EPAD, 16), _f32),
                  jax.ShapeDtypeStruct((_NP, 16), _f32),
                  jax.ShapeDtypeStruct((_NP, 16), _f32)],
        mesh=_mesh(),
        compiler_params=pltpu.CompilerParams(use_tc_tiling_on_sc=False),
        scratch_types=[
            pltpu.VMEM((_CHUNK,), jnp.int32),
            pltpu.VMEM((_CHUNK,), jnp.int32),
            pltpu.VMEM((_CHUNK, 16), _f32),
            pltpu.VMEM((_CHUNK, 16), _f32),
            pltpu.VMEM((_CHUNK, 16), _f32),
            pltpu.VMEM((_RPT, 16), _f32),
            pltpu.VMEM_SHARED((_NP, 16), _f32),
            pltpu.SemaphoreType.DMA,
        ],
    )(T, src, dst)


def _sc_b_body(src_hbm, dst_hbm, ex_hbm, den0_hbm, den1_hbm,
               lo_hbm, hi_hbm,
               o0l_hbm, o1l_hbm, o0h_hbm, o1h_hbm,
               sidx, didx, exb, d0, d1, alb, rows, msg, zbuf, out_sh, sem):
    c = lax.axis_index("c")
    s = lax.axis_index("s")
    wid = c * _NS + s
    hw = _HID // 2

    for half, (hp_hbm, oa_hbm, ob_hbm) in enumerate(
            [(lo_hbm, o0l_hbm, o1l_hbm), (hi_hbm, o0h_hbm, o1h_hbm)]):
        def zrow(r, carry):
            for cs in range(hw // 16):
                zbuf[r, cs * 16:(cs + 1) * 16] = jnp.zeros((16,), _f32)
            return carry
        lax.fori_loop(0, 128, zrow, 0)
        for j in range(5):
            pltpu.sync_copy(zbuf,
                            out_sh.at[pl.ds(s * _RPT + j * 128, 128), :])
        plsc.subcore_barrier()

        def chunk_body(k, carry):
            base = wid * _EPT + k * _CHUNK
            pltpu.sync_copy(src_hbm.at[pl.ds(base, _CHUNK)], sidx)
            pltpu.sync_copy(dst_hbm.at[pl.ds(base, _CHUNK)], didx)
            pltpu.sync_copy(ex_hbm.at[pl.ds(base, _CHUNK), :], exb)
            pltpu.async_copy(hp_hbm.at[sidx], rows, sem).wait()
            pltpu.async_copy(den0_hbm.at[didx], d0, sem).wait()
            pltpu.async_copy(den1_hbm.at[didx], d1, sem).wait()

            def alpha(e, c2):
                alb[e, :] = (exb[e, :]
                             / (d0[e, :] + d1[e, :] + 1e-16) * 0.125)
                return c2
            lax.fori_loop(0, _CHUNK, alpha, 0)

            def message(e, c2):
                accs = [jnp.zeros((16,), _f32) for _ in range(hw // 16)]
                arow = alb[e, :]
                for h in range(_HEADS):
                    ah = _vgather16(arow, jnp.full((16,), h, jnp.int32))
                    for cs in range(hw // 16):
                        off = h * hw + cs * 16
                        accs[cs] = accs[cs] + ah * rows[e, off:off + 16]
                for cs in range(hw // 16):
                    msg[e, cs * 16:(cs + 1) * 16] = accs[cs]
                return c2
            lax.fori_loop(0, _CHUNK, message, 0)
            pltpu.sync_copy(msg, out_sh.at[didx], add=True)
            return carry
        lax.fori_loop(0, _CPT, chunk_body, 0)
        plsc.subcore_barrier()

        @pl.when(c == 0)
        def _():
            pltpu.sync_copy(out_sh.at[pl.ds(s * _RPT, _RPT), :],
                            oa_hbm.at[pl.ds(s * _RPT, _RPT), :])

        @pl.when(c == 1)
        def _():
            pltpu.sync_copy(out_sh.at[pl.ds(s * _RPT, _RPT), :],
                            ob_hbm.at[pl.ds(s * _RPT, _RPT), :])
        plsc.subcore_barrier()


def _sc_pass_b(src, dst, ex, den0, den1, hp_lo, hp_hi):
    hw = _HID // 2
    return pl.kernel(
        _sc_b_body,
        out_type=[jax.ShapeDtypeStruct((_NP, hw), _f32),
                  jax.ShapeDtypeStruct((_NP, hw), _f32),
                  jax.ShapeDtypeStruct((_NP, hw), _f32),
                  jax.ShapeDtypeStruct((_NP, hw), _f32)],
        mesh=_mesh(),
        compiler_params=pltpu.CompilerParams(use_tc_tiling_on_sc=False),
        scratch_types=[
            pltpu.VMEM((_CHUNK,), jnp.int32),
            pltpu.VMEM((_CHUNK,), jnp.int32),
            pltpu.VMEM((_CHUNK, 16), _f32),
            pltpu.VMEM((_CHUNK, 16), _f32),
            pltpu.VMEM((_CHUNK, 16), _f32),
            pltpu.VMEM((_CHUNK, 16), _f32),
            pltpu.VMEM((_CHUNK, _HH // 2), _f32),
            pltpu.VMEM((_CHUNK, _HID // 2), _f32),
            pltpu.VMEM((128, _HID // 2), _f32),
            pltpu.VMEM_SHARED((_NP, _HID // 2), _f32),
            pltpu.SemaphoreType.DMA,
        ],
    )(src, dst, ex, den0, den1, hp_lo, hp_hi)


@jax.jit
def kernel(x, edge_index, W_in, b_in, W_gat, att_src, att_dst, b_gat,
           bn_gamma, bn_beta, W1, b1, W2, b2):
    loops = jnp.arange(_N, dtype=jnp.int32)
    pad = jnp.zeros((_EPAD - _ETOT,), jnp.int32)
    srcp = jnp.concatenate([edge_index[0].astype(jnp.int32), loops, pad])
    dstp = jnp.concatenate([edge_index[1].astype(jnp.int32), loops, pad])

    eye = jnp.eye(_HEADS, dtype=_f32)
    A_s = (att_src[:, :, :, None] * eye[:, None, :]).reshape(_L, _HH, _HEADS)
    A_d = (att_dst[:, :, :, None] * eye[:, None, :]).reshape(_L, _HH, _HEADS)
    A = jnp.concatenate([A_s, A_d], axis=-1)
    Wt = jnp.einsum('lij,ljk->lik', W_gat, A)

    W4 = W_gat.reshape(_L, _HID, _HEADS, _HID)
    W_lo = W4[:, :, :, :_HID // 2].reshape(_L, _HID, _HH // 2)
    W_hi = W4[:, :, :, _HID // 2:].reshape(_L, _HID, _HH // 2)

    W2p = jnp.zeros((_HID // 2, _HID), _f32).at[:, :2].set(W2)
    b2p = jnp.zeros((_HID,), _f32).at[:2].set(b2)

    h0 = _tc_in(x, W_in, b_in)

    def step(h, xs):
        wlo, whi, wt, bg, ga, be = xs
        hp_lo, hp_hi, T = _tc_pre(h, wlo, whi, wt)
        ex, den0, den1 = _sc_pass_a(T, srcp, dstp)
        o0l, o1l, o0h, o1h = _sc_pass_b(srcp, dstp, ex, den0, den1,
                                        hp_lo, hp_hi)
        h2 = _tc_post(o0l, o1l, o0h, o1h, h, bg, ga, be)
        return h2, None

    h, _ = lax.scan(step, h0, (W_lo, W_hi, Wt, b_gat, bn_gamma, bn_beta))
    out = _tc_head(h, W1, b1, W2p, b2p)
    return out[:, :2]
